# Initial kernel scaffold; baseline (speedup 1.0000x reference)
import jax
import jax.numpy as jnp
from jax.experimental import pallas as pl


def kernel(z, edge_idx):
    raise NotImplementedError("write your pallas kernel here")


# speedup vs baseline: 1.3688x; 1.3688x over previous
import functools

import jax
import jax.numpy as jnp
from jax import lax
from jax.experimental import pallas as pl
from jax.experimental.pallas import tpu as pltpu
from jax.experimental.pallas import tpu_sc as plsc

E = 160000
D = 256
L = 16
NC, NS = 2, 16
NW = NC * NS
EP = 163840
PER_W = EP // NW
B = 128
NBLK = PER_W // B
DV = D // L

_mesh = plsc.VectorSubcoreMesh(core_axis_name="c", subcore_axis_name="s")

_GATHER_DN = lax.GatherDimensionNumbers(
    offset_dims=(), collapsed_slice_dims=(0,), start_index_map=(0,))


def _rotate(v, perm):
    return lax.gather(v, perm[:, None], _GATHER_DN, slice_sizes=(1,),
                      mode=lax.GatherScatterMode.PROMISE_IN_BOUNDS)


@functools.partial(
    pl.kernel,
    mesh=_mesh,
    out_type=jax.ShapeDtypeStruct((EP,), jnp.float32),
    scratch_types=[
        pltpu.VMEM((B,), jnp.int32),
        pltpu.VMEM((B,), jnp.int32),
        pltpu.VMEM((B, D), jnp.float32),
        pltpu.VMEM((B, D), jnp.float32),
        pltpu.VMEM((B,), jnp.float32),
        pltpu.SemaphoreType.DMA,
        pltpu.SemaphoreType.DMA,
    ],
)
def _decode(z_hbm, sidx_hbm, didx_hbm, out_hbm,
            sidx_v, didx_v, srows_v, drows_v, outb_v, sem_s, sem_d):
    wid = lax.axis_index("s") * NC + lax.axis_index("c")
    wbase = wid * PER_W

    def blk_body(b, carry):
        base = wbase + b * B
        pltpu.sync_copy(sidx_hbm.at[pl.ds(base, B)], sidx_v)
        pltpu.sync_copy(didx_hbm.at[pl.ds(base, B)], didx_v)
        cp_s = pltpu.async_copy(z_hbm.at[sidx_v], srows_v, sem_s)
        cp_d = pltpu.async_copy(z_hbm.at[didx_v], drows_v, sem_d)
        cp_s.wait()
        cp_d.wait()

        lanes = lax.broadcasted_iota(jnp.int32, (L,), 0)
        rots = [(lanes + r) % L for r in (8, 4, 2, 1)]

        def grp_body(g, c):
            gbase = g * L

            def edge_body(i, res):
                e = gbase + i
                acc = srows_v[e, pl.ds(0, L)] * drows_v[e, pl.ds(0, L)]
                for j in range(1, DV):
                    acc = acc + (srows_v[e, pl.ds(j * L, L)]
                                 * drows_v[e, pl.ds(j * L, L)])
                for perm in rots:
                    acc = acc + _rotate(acc, perm)
                return lax.select(lanes == i, acc, res)

            res = lax.fori_loop(0, L, edge_body,
                                jnp.zeros((L,), jnp.float32), unroll=2)
            res = 1.0 / (1.0 + jnp.exp(-res))
            outb_v[pl.ds(pl.multiple_of(gbase, L), L)] = res
            return c

        lax.fori_loop(0, B // L, grp_body, 0)

        pltpu.sync_copy(outb_v, out_hbm.at[pl.ds(base, B)])
        return carry

    lax.fori_loop(0, NBLK, blk_body, 0)


def kernel(z, edge_idx):
    idx = edge_idx.astype(jnp.int32)
    pad = EP - E
    sidx = jnp.pad(idx[0], (0, pad))
    didx = jnp.pad(idx[1], (0, pad))
    out = _decode(z, sidx, didx)
    return out[:E]


# speedup vs baseline: 1.6785x; 1.2262x over previous
import functools

import jax
import jax.numpy as jnp
from jax import lax
from jax.experimental import pallas as pl
from jax.experimental.pallas import tpu as pltpu
from jax.experimental.pallas import tpu_sc as plsc

E = 160000
D = 256
L = 16
NC, NS = 2, 16
NW = NC * NS
EP = 163840
PER_W = EP // NW
B = 64
NBLK = PER_W // B
DV = D // L

_mesh = plsc.VectorSubcoreMesh(core_axis_name="c", subcore_axis_name="s")

_GATHER_DN = lax.GatherDimensionNumbers(
    offset_dims=(), collapsed_slice_dims=(0,), start_index_map=(0,))


def _rotate(v, perm):
    return lax.gather(v, perm[:, None], _GATHER_DN, slice_sizes=(1,),
                      mode=lax.GatherScatterMode.PROMISE_IN_BOUNDS)


@functools.partial(
    pl.kernel,
    mesh=_mesh,
    out_type=jax.ShapeDtypeStruct((EP,), jnp.float32),
    scratch_types=[
        pltpu.VMEM((PER_W,), jnp.int32),
        pltpu.VMEM((PER_W,), jnp.int32),
        pltpu.VMEM((B, D), jnp.float32),
        pltpu.VMEM((B, D), jnp.float32),
        pltpu.VMEM((B, D), jnp.float32),
        pltpu.VMEM((B, D), jnp.float32),
        pltpu.VMEM((B,), jnp.float32),
        pltpu.VMEM((B,), jnp.float32),
        pltpu.SemaphoreType.DMA,
        pltpu.SemaphoreType.DMA,
    ],
)
def _decode(z_hbm, sidx_hbm, didx_hbm, out_hbm,
            sidx_v, didx_v, sr0, dr0, sr1, dr1, ob0, ob1, sem0, sem1):
    wid = lax.axis_index("s") * NC + lax.axis_index("c")
    wbase = wid * PER_W
    slots = ((sr0, dr0, ob0, sem0), (sr1, dr1, ob1, sem1))

    pltpu.sync_copy(sidx_hbm.at[pl.ds(wbase, PER_W)], sidx_v)
    pltpu.sync_copy(didx_hbm.at[pl.ds(wbase, PER_W)], didx_v)

    lanes = lax.broadcasted_iota(jnp.int32, (L,), 0)
    rots = [(lanes + r) % L for r in (8, 4, 2, 1)]

    def issue(blk, sr, dr, sem):
        base = blk * B
        pltpu.async_copy(z_hbm.at[sidx_v.at[pl.ds(base, B)]], sr, sem)
        pltpu.async_copy(z_hbm.at[didx_v.at[pl.ds(base, B)]], dr, sem)

    issue(0, sr0, dr0, sem0)
    issue(1, sr1, dr1, sem1)

    def body(g, c):
        for s, (sr, dr, ob, sem) in enumerate(slots):
            blk = 2 * g + s
            base = blk * B
            pltpu.make_async_copy(
                z_hbm.at[sidx_v.at[pl.ds(base, B)]], sr, sem).wait()
            pltpu.make_async_copy(
                z_hbm.at[didx_v.at[pl.ds(base, B)]], dr, sem).wait()

            def grp_body(g2, c2):
                gbase = g2 * L

                def edge_body(i, res):
                    e = gbase + i
                    acc = sr[e, pl.ds(0, L)] * dr[e, pl.ds(0, L)]
                    for j in range(1, DV):
                        acc = acc + (sr[e, pl.ds(j * L, L)]
                                     * dr[e, pl.ds(j * L, L)])
                    for perm in rots:
                        acc = acc + _rotate(acc, perm)
                    return lax.select(lanes == i, acc, res)

                res = lax.fori_loop(0, L, edge_body,
                                    jnp.zeros((L,), jnp.float32), unroll=2)
                res = 1.0 / (1.0 + jnp.exp(-res))
                ob[pl.ds(pl.multiple_of(gbase, L), L)] = res
                return c2

            lax.fori_loop(0, B // L, grp_body, 0)

            @pl.when(blk + 2 < NBLK)
            def _():
                issue(blk + 2, sr, dr, sem)

            pltpu.sync_copy(ob, out_hbm.at[pl.ds(wbase + base, B)])
        return c

    lax.fori_loop(0, NBLK // 2, body, 0)


def kernel(z, edge_idx):
    idx = edge_idx.astype(jnp.int32)
    pad = EP - E
    sidx = jnp.pad(idx[0], (0, pad))
    didx = jnp.pad(idx[1], (0, pad))
    out = _decode(z, sidx, didx)
    return out[:E]


# speedup vs baseline: 1.9931x; 1.1874x over previous
import functools

import jax
import jax.numpy as jnp
from jax import lax
from jax.experimental import pallas as pl
from jax.experimental.pallas import tpu as pltpu
from jax.experimental.pallas import tpu_sc as plsc

E = 160000
D = 256
L = 16
NC, NS = 2, 16
EP = 163840
PAIR_W = EP // NS
B = 64
NBLK_PAIR = PAIR_W // B
NBLK_FAST = 120
NBLK_SLOW = NBLK_PAIR - NBLK_FAST
DV = D // L

_mesh = plsc.VectorSubcoreMesh(core_axis_name="c", subcore_axis_name="s")

_GATHER_DN = lax.GatherDimensionNumbers(
    offset_dims=(), collapsed_slice_dims=(0,), start_index_map=(0,))


def _rotate(v, perm):
    return lax.gather(v, perm[:, None], _GATHER_DN, slice_sizes=(1,),
                      mode=lax.GatherScatterMode.PROMISE_IN_BOUNDS)


@functools.partial(
    pl.kernel,
    mesh=_mesh,
    out_type=jax.ShapeDtypeStruct((EP,), jnp.float32),
    scratch_types=[
        pltpu.VMEM((NBLK_FAST * B,), jnp.int32),
        pltpu.VMEM((NBLK_FAST * B,), jnp.int32),
        pltpu.VMEM((B, D), jnp.float32),
        pltpu.VMEM((B, D), jnp.float32),
        pltpu.VMEM((B, D), jnp.float32),
        pltpu.VMEM((B, D), jnp.float32),
        pltpu.VMEM((B,), jnp.float32),
        pltpu.VMEM((B,), jnp.float32),
        pltpu.SemaphoreType.DMA,
        pltpu.SemaphoreType.DMA,
    ],
)
def _decode(z_hbm, sidx_hbm, didx_hbm, out_hbm,
            sidx_v, didx_v, sr0, dr0, sr1, dr1, ob0, ob1, sem0, sem1):
    cid = lax.axis_index("c")
    sid = lax.axis_index("s")
    wbase = sid * PAIR_W + cid * (NBLK_FAST * B)
    nblk = jnp.where(cid == 0, NBLK_FAST, NBLK_SLOW)
    nedge = nblk * B
    slots = ((sr0, dr0, ob0, sem0), (sr1, dr1, ob1, sem1))

    @pl.when(cid == 0)
    def _():
        pltpu.sync_copy(sidx_hbm.at[pl.ds(wbase, NBLK_FAST * B)], sidx_v)
        pltpu.sync_copy(didx_hbm.at[pl.ds(wbase, NBLK_FAST * B)], didx_v)

    @pl.when(cid != 0)
    def _():
        pltpu.sync_copy(sidx_hbm.at[pl.ds(wbase, NBLK_SLOW * B)],
                        sidx_v.at[pl.ds(0, NBLK_SLOW * B)])
        pltpu.sync_copy(didx_hbm.at[pl.ds(wbase, NBLK_SLOW * B)],
                        didx_v.at[pl.ds(0, NBLK_SLOW * B)])

    lanes = lax.broadcasted_iota(jnp.int32, (L,), 0)
    rots = [(lanes + r) % L for r in (8, 4, 2, 1)]

    def issue(blk, sr, dr, sem):
        base = blk * B
        pltpu.async_copy(z_hbm.at[sidx_v.at[pl.ds(base, B)]], sr, sem)
        pltpu.async_copy(z_hbm.at[didx_v.at[pl.ds(base, B)]], dr, sem)

    issue(0, sr0, dr0, sem0)
    issue(1, sr1, dr1, sem1)

    def body(g, c):
        for s, (sr, dr, ob, sem) in enumerate(slots):
            blk = 2 * g + s
            base = blk * B
            pltpu.make_async_copy(
                z_hbm.at[sidx_v.at[pl.ds(base, B)]], sr, sem).wait()
            pltpu.make_async_copy(
                z_hbm.at[didx_v.at[pl.ds(base, B)]], dr, sem).wait()

            def grp_body(g2, c2):
                gbase = g2 * L

                def edge_body(i, res):
                    e = gbase + i
                    acc = sr[e, pl.ds(0, L)] * dr[e, pl.ds(0, L)]
                    for j in range(1, DV):
                        acc = acc + (sr[e, pl.ds(j * L, L)]
                                     * dr[e, pl.ds(j * L, L)])
                    for perm in rots:
                        acc = acc + _rotate(acc, perm)
                    return lax.select(lanes == i, acc, res)

                res = lax.fori_loop(0, L, edge_body,
                                    jnp.zeros((L,), jnp.float32), unroll=2)
                res = 1.0 / (1.0 + jnp.exp(-res))
                ob[pl.ds(pl.multiple_of(gbase, L), L)] = res
                return c2

            lax.fori_loop(0, B // L, grp_body, 0)

            @pl.when(blk + 2 < nblk)
            def _():
                issue(blk + 2, sr, dr, sem)

            pltpu.sync_copy(ob, out_hbm.at[pl.ds(wbase + base, B)])
        return c

    lax.fori_loop(0, nblk // 2, body, 0)


def kernel(z, edge_idx):
    idx = edge_idx.astype(jnp.int32)
    pad = EP - E
    sidx = jnp.pad(idx[0], (0, pad))
    didx = jnp.pad(idx[1], (0, pad))
    out = _decode(z, sidx, didx)
    return out[:E]
